# Initial kernel scaffold; baseline (speedup 1.0000x reference)
import functools
import jax
import jax.numpy as jnp
from jax import lax
from jax.experimental import pallas as pl
from jax.experimental.pallas import tpu as pltpu
from jax.experimental.pallas import tpu_sc as plsc

N = 10016
NT = 32
NPT = N // NT
C = 128


def _probe_call(srcs, dls, evs, feat):
    E = srcs.shape[0]
    mesh = plsc.VectorSubcoreMesh(core_axis_name="c", subcore_axis_name="s")

    @functools.partial(
        pl.kernel,
        out_type=jax.ShapeDtypeStruct((N, 128), jnp.float32),
        mesh=mesh,
        scratch_types=[
            pltpu.VMEM((C,), jnp.int32),
            pltpu.VMEM((C,), jnp.int32),
            pltpu.VMEM((C, 16), jnp.float32),
            pltpu.VMEM((C, 128), jnp.float32),
            pltpu.VMEM((NPT, 128), jnp.float32),
            pltpu.SemaphoreType.DMA,
        ],
    )
    def k(src_hbm, dl_hbm, ev_hbm, feat_hbm, out_hbm,
          idx_v, dl_v, ev_v, rows_v, acc_v, sem):
        cid = lax.axis_index("c")
        sid = lax.axis_index("s")
        wid = sid * 2 + cid
        nchunks = E // (NT * C)

        def zrow(i, _):
            acc_v[i] = jnp.zeros((128,), jnp.float32)
            return 0
        lax.fori_loop(0, NPT, zrow, 0)

        def chunk(i, _):
            ec = wid * (E // NT) + i * C
            pltpu.sync_copy(src_hbm.at[pl.ds(ec, C)], idx_v)
            pltpu.sync_copy(dl_hbm.at[pl.ds(ec, C)], dl_v)
            pltpu.sync_copy(ev_hbm.at[pl.ds(ec, C), :], ev_v)
            pltpu.async_copy(feat_hbm.at[idx_v], rows_v, sem).wait()

            def edge(e, _):
                dl = dl_v[e]
                for j in range(8):
                    s = ev_v[e, j]
                    evj = jnp.full((16,), s, jnp.float32)
                    r = rows_v[e, pl.ds(16 * j, 16)]
                    acc_v[dl, pl.ds(16 * j, 16)] += jnp.exp(r * evj)
                return 0

            lax.fori_loop(0, C, edge, 0)
            return 0

        lax.fori_loop(0, nchunks, chunk, 0)
        pltpu.sync_copy(acc_v, out_hbm.at[pl.ds(wid * NPT, NPT), :])

    return k(srcs, dls, evs, feat)


def kernel(ent_ids, rel_ids, edge_index, params):
    src = edge_index[0]
    dst = edge_index[1]
    E = src.shape[0]
    perm = jnp.argsort(dst)
    src_s = src[perm]
    dst_s = dst[perm]
    dl = dst_s % NPT
    evs = jnp.zeros((E, 16), jnp.float32)
    feat = jnp.pad(params["node_table"], ((0, N - 10000), (0, 0)))
    out = _probe_call(src_s, dl, evs, feat)
    return out[jnp.arange(8) * (10000 // 8)]


# speedup vs baseline: 32.6793x; 32.6793x over previous
import functools
import jax
import jax.numpy as jnp
import numpy as np
from jax import lax
from jax.experimental import pallas as pl
from jax.experimental.pallas import tpu as pltpu
from jax.experimental.pallas import tpu_sc as plsc

NODES = 10000
N = 10240
NT = 32
NPT = N // NT
C = 128
HEADS = 8
DH = 16
HID = 128
ALPHA = 0.15
HOPS = 4
SQRT_DH = 4.0

_MESH = plsc.VectorSubcoreMesh(core_axis_name="c", subcore_axis_name="s")


def _wid():
    return lax.axis_index("s") * 2 + lax.axis_index("c")


def _m8(x):
    return pl.multiple_of(x, 8)


def _tile_info(ti_v):
    row = ti_v[...]
    return row[0], row[1], row[2], row[3]


def _proj_body(x_ref, w_ref, q_ref, k_ref, v_ref, r_ref):
    y = jnp.dot(x_ref[...], w_ref[...], preferred_element_type=jnp.float32)
    q_ref[...] = y[:, 0:128]
    k_ref[...] = y[:, 128:256]
    v_ref[...] = y[:, 256:384]
    r_ref[...] = y[:, 384:512]


def _proj(h, wstack):
    grid = (N // 256,)
    out = jax.ShapeDtypeStruct((N, HID), jnp.float32)
    return pl.pallas_call(
        _proj_body,
        grid=grid,
        in_specs=[
            pl.BlockSpec((256, HID), lambda i: (i, 0)),
            pl.BlockSpec((HID, 512), lambda i: (0, 0)),
        ],
        out_specs=[pl.BlockSpec((256, HID), lambda i: (i, 0))] * 4,
        out_shape=[out] * 4,
    )(h, wstack)


def _rel_body(t_ref, w_ref, o_ref):
    o_ref[...] = jnp.dot(t_ref[...], w_ref[...],
                         preferred_element_type=jnp.float32)


def _rel_proj(rel_table, wr):
    t = jnp.pad(rel_table, ((0, 0), (0, 112)))
    w = jnp.pad(wr, ((0, 112), (0, 0)))
    return pl.pallas_call(
        _rel_body,
        out_shape=jax.ShapeDtypeStruct((16, HID), jnp.float32),
    )(t, w)


def _combine_body(f_ref, r_ref, o_ref):
    x = f_ref[...] + r_ref[...]
    o_ref[...] = jnp.where(x > 0, x, jnp.exp(jnp.minimum(x, 0.0)) - 1.0)


def _combine(feat, res):
    return pl.pallas_call(
        _combine_body,
        grid=(N // 256,),
        in_specs=[pl.BlockSpec((256, HID), lambda i: (i, 0))] * 2,
        out_specs=pl.BlockSpec((256, HID), lambda i: (i, 0)),
        out_shape=jax.ShapeDtypeStruct((N, HID), jnp.float32),
    )(feat, res)


@functools.lru_cache(maxsize=None)
def _make_scores(EP, with_rel):
    scratch = [
        pltpu.VMEM((16,), jnp.int32),
        pltpu.VMEM((C,), jnp.int32),
        pltpu.VMEM((C + 16,), jnp.int32),
        pltpu.VMEM((C + 16,), jnp.int32),
        pltpu.VMEM((C, HID), jnp.float32),
        pltpu.VMEM((C * 16,), jnp.float32),
        pltpu.VMEM((NPT, HID), jnp.float32),
        pltpu.VMEM((16, HID), jnp.float32),
        pltpu.VMEM((NPT * 16,), jnp.float32),
        pltpu.SemaphoreType.DMA,
    ]

    @functools.partial(
        pl.kernel,
        out_type=[
            jax.ShapeDtypeStruct((EP * 16,), jnp.float32),
            jax.ShapeDtypeStruct((N * 16,), jnp.float32),
        ],
        mesh=_MESH,
        scratch_types=scratch,
    )
    def scores_kernel(ti_hbm, src_hbm, dl_hbm, rel_hbm, q_hbm, k_hbm, r_hbm,
                      sc_hbm, mx_hbm,
                      ti_v, idx_v, dl_v, rel_v, rows_v, sc_v, q_v, r_v, mx_v,
                      sem):
        w = _wid()
        pltpu.sync_copy(ti_hbm.at[pl.ds(_m8(w * 16), 16)], ti_v)
        e0, e1, base, nch = _tile_info(ti_v)
        pltpu.sync_copy(q_hbm.at[pl.ds(_m8(w * NPT), NPT), :], q_v)
        pltpu.sync_copy(r_hbm, r_v)

        neg = jnp.full((16,), -3.0e38, jnp.float32)

        def zrow(i, _):
            mx_v[pl.ds(i * 16, 16)] = neg
            return 0
        lax.fori_loop(0, NPT, zrow, 0)

        def chunk(i, _):
            cs = base + i * C
            pltpu.sync_copy(src_hbm.at[pl.ds(_m8(cs), C)], idx_v)
            pltpu.sync_copy(dl_hbm.at[pl.ds(_m8(cs), C)], dl_v.at[pl.ds(0, C)])
            if with_rel:
                pltpu.sync_copy(rel_hbm.at[pl.ds(_m8(cs), C)],
                                rel_v.at[pl.ds(0, C)])
            pltpu.async_copy(k_hbm.at[idx_v], rows_v, sem).wait()

            lo = jnp.maximum(e0 - cs, 0)
            hi = jnp.minimum(e1 - cs, C)

            def edge(e, _):
                dl = dl_v[pl.ds(e, 16)][0]
                srow = jnp.zeros((16,), jnp.float32)
                iota = lax.iota(jnp.int32, 16)
                if with_rel:
                    rel = rel_v[pl.ds(e, 16)][0]
                for j in range(HEADS):
                    kj = rows_v[e, pl.ds(16 * j, 16)]
                    if with_rel:
                        kj = kj + r_v[rel, pl.ds(16 * j, 16)]
                    qj = q_v[dl, pl.ds(16 * j, 16)]
                    p = qj * kj
                    for sh in (1, 2, 4, 8):
                        p = p + jnp.take_along_axis(
                            p, jnp.bitwise_xor(iota, sh), axis=0)
                    srow = jnp.where(iota == j, p, srow)
                srow = srow * (1.0 / SQRT_DH)
                srow = jnp.where(srow > 0, srow, 0.2 * srow)
                sc_v[pl.ds(e * 16, 16)] = srow
                m = mx_v[pl.ds(dl * 16, 16)]
                mx_v[pl.ds(dl * 16, 16)] = jnp.maximum(m, srow)
                return 0

            lax.fori_loop(lo, hi, edge, 0)

            full = jnp.logical_and(cs >= e0, cs + C <= e1)

            @pl.when(full)
            def _():
                pltpu.sync_copy(sc_v, sc_hbm.at[pl.ds(_m8(cs * 16), C * 16)])

            @pl.when(jnp.logical_not(full))
            def _():
                def wr(e, _):
                    pltpu.sync_copy(sc_v.at[pl.ds(e * 16, 16)],
                                    sc_hbm.at[pl.ds(_m8((cs + e) * 16), 16)])
                    return 0
                lax.fori_loop(lo, hi, wr, 0)
            return 0

        lax.fori_loop(0, nch, chunk, 0)
        pltpu.sync_copy(mx_v, mx_hbm.at[pl.ds(_m8(w * NPT * 16), NPT * 16)])

    return scores_kernel


@functools.lru_cache(maxsize=None)
def _make_expsum(EP):
    scratch = [
        pltpu.VMEM((16,), jnp.int32),
        pltpu.VMEM((C + 16,), jnp.int32),
        pltpu.VMEM((C * 16,), jnp.float32),
        pltpu.VMEM((NPT * 16,), jnp.float32),
        pltpu.VMEM((NPT * 16,), jnp.float32),
    ]

    @functools.partial(
        pl.kernel,
        out_type=[
            jax.ShapeDtypeStruct((EP * 16,), jnp.float32),
            jax.ShapeDtypeStruct((N * 16,), jnp.float32),
        ],
        mesh=_MESH,
        scratch_types=scratch,
    )
    def expsum_kernel(ti_hbm, dl_hbm, sc_hbm, mx_hbm, ev_hbm, inv_hbm,
                      ti_v, dl_v, sc_v, mx_v, st_v):
        w = _wid()
        pltpu.sync_copy(ti_hbm.at[pl.ds(_m8(w * 16), 16)], ti_v)
        e0, e1, base, nch = _tile_info(ti_v)
        pltpu.sync_copy(mx_hbm.at[pl.ds(_m8(w * NPT * 16), NPT * 16)], mx_v)

        zero = jnp.zeros((16,), jnp.float32)

        def zrow(i, _):
            st_v[pl.ds(i * 16, 16)] = zero
            return 0
        lax.fori_loop(0, NPT, zrow, 0)

        def chunk(i, _):
            cs = base + i * C
            pltpu.sync_copy(dl_hbm.at[pl.ds(_m8(cs), C)], dl_v.at[pl.ds(0, C)])
            pltpu.sync_copy(sc_hbm.at[pl.ds(_m8(cs * 16), C * 16)], sc_v)

            lo = jnp.maximum(e0 - cs, 0)
            hi = jnp.minimum(e1 - cs, C)

            def edge(e, _):
                dl = dl_v[pl.ds(e, 16)][0]
                srow = sc_v[pl.ds(e * 16, 16)]
                m = mx_v[pl.ds(dl * 16, 16)]
                ex = jnp.exp(srow - m)
                sc_v[pl.ds(e * 16, 16)] = ex
                st_v[pl.ds(dl * 16, 16)] += ex
                return 0

            lax.fori_loop(lo, hi, edge, 0)

            full = jnp.logical_and(cs >= e0, cs + C <= e1)

            @pl.when(full)
            def _():
                pltpu.sync_copy(sc_v, ev_hbm.at[pl.ds(_m8(cs * 16), C * 16)])

            @pl.when(jnp.logical_not(full))
            def _():
                def wr(e, _):
                    pltpu.sync_copy(sc_v.at[pl.ds(e * 16, 16)],
                                    ev_hbm.at[pl.ds(_m8((cs + e) * 16), 16)])
                    return 0
                lax.fori_loop(lo, hi, wr, 0)
            return 0

        lax.fori_loop(0, nch, chunk, 0)

        def inv(i, _):
            s = st_v[pl.ds(i * 16, 16)]
            st_v[pl.ds(i * 16, 16)] = 1.0 / (s + 1e-9)
            return 0
        lax.fori_loop(0, NPT, inv, 0)
        pltpu.sync_copy(st_v, inv_hbm.at[pl.ds(_m8(w * NPT * 16), NPT * 16)])

    return expsum_kernel


@functools.lru_cache(maxsize=None)
def _make_hop(EP):
    scratch = [
        pltpu.VMEM((16,), jnp.int32),
        pltpu.VMEM((C,), jnp.int32),
        pltpu.VMEM((C + 16,), jnp.int32),
        pltpu.VMEM((C * 16,), jnp.float32),
        pltpu.VMEM((C, HID), jnp.float32),
        pltpu.VMEM((NPT, HID), jnp.float32),
        pltpu.VMEM((NPT, HID), jnp.float32),
        pltpu.VMEM((NPT * 16,), jnp.float32),
        pltpu.SemaphoreType.DMA,
    ]

    @functools.partial(
        pl.kernel,
        out_type=jax.ShapeDtypeStruct((N, HID), jnp.float32),
        mesh=_MESH,
        scratch_types=scratch,
    )
    def hop_kernel(ti_hbm, src_hbm, dl_hbm, ev_hbm, inv_hbm, feat_hbm, v_hbm,
                   out_hbm,
                   ti_v, idx_v, dl_v, ev_v, rows_v, acc_v, v0_v, inv_v, sem):
        w = _wid()
        pltpu.sync_copy(ti_hbm.at[pl.ds(_m8(w * 16), 16)], ti_v)
        e0, e1, base, nch = _tile_info(ti_v)
        pltpu.sync_copy(inv_hbm.at[pl.ds(_m8(w * NPT * 16), NPT * 16)], inv_v)
        pltpu.sync_copy(v_hbm.at[pl.ds(_m8(w * NPT), NPT), :], v0_v)

        zero = jnp.zeros((16,), jnp.float32)

        def zrow(i, _):
            for j in range(HEADS):
                acc_v[i, pl.ds(16 * j, 16)] = zero
            return 0
        lax.fori_loop(0, NPT, zrow, 0)

        def chunk(i, _):
            cs = base + i * C
            pltpu.sync_copy(src_hbm.at[pl.ds(_m8(cs), C)], idx_v)
            pltpu.sync_copy(dl_hbm.at[pl.ds(_m8(cs), C)], dl_v.at[pl.ds(0, C)])
            pltpu.sync_copy(ev_hbm.at[pl.ds(_m8(cs * 16), C * 16)], ev_v)
            pltpu.async_copy(feat_hbm.at[idx_v], rows_v, sem).wait()

            lo = jnp.maximum(e0 - cs, 0)
            hi = jnp.minimum(e1 - cs, C)

            def edge(e, _):
                dl = dl_v[pl.ds(e, 16)][0]
                evrow = ev_v[pl.ds(e * 16, 16)]
                for j in range(HEADS):
                    evj = jnp.full((16,), evrow[j], jnp.float32)
                    r = rows_v[e, pl.ds(16 * j, 16)]
                    acc_v[dl, pl.ds(16 * j, 16)] += r * evj
                return 0

            lax.fori_loop(lo, hi, edge, 0)
            return 0

        lax.fori_loop(0, nch, chunk, 0)

        def finish(i, _):
            invrow = inv_v[pl.ds(i * 16, 16)]
            for j in range(HEADS):
                ivj = jnp.full((16,), invrow[j], jnp.float32)
                a = acc_v[i, pl.ds(16 * j, 16)]
                f0 = v0_v[i, pl.ds(16 * j, 16)]
                acc_v[i, pl.ds(16 * j, 16)] = (1.0 - ALPHA) * a * ivj + ALPHA * f0
            return 0
        lax.fori_loop(0, NPT, finish, 0)
        pltpu.sync_copy(acc_v, out_hbm.at[pl.ds(_m8(w * NPT), NPT), :])

    return hop_kernel


def _layer(h, wstack, src_s, dl, tinfo, EP, rel_s=None, rtab=None):
    q, k, v, res = _proj(h, wstack)
    with_rel = rel_s is not None
    if not with_rel:
        rtab = jnp.zeros((16, HID), jnp.float32)
        rel_s = jnp.zeros((EP,), jnp.int32)
    scores_k = _make_scores(EP, with_rel)
    sc, mx = scores_k(tinfo, src_s, dl, rel_s, q, k, rtab)
    expsum_k = _make_expsum(EP)
    ev, inv = expsum_k(tinfo, dl, sc, mx)
    hop_k = _make_hop(EP)
    feat = v
    for _ in range(HOPS):
        feat = hop_k(tinfo, src_s, dl, ev, inv, feat, v)
    return _combine(feat, res)


def kernel(ent_ids, rel_ids, edge_index, params):
    src = edge_index[0].astype(jnp.int32)
    dst = edge_index[1].astype(jnp.int32)
    E = src.shape[0]
    EP = E + C

    perm = jnp.argsort(dst)
    src_s = jnp.pad(src[perm], (0, EP - E))
    dst_s = dst[perm]
    rel_s = jnp.pad(rel_ids[perm].astype(jnp.int32), (0, EP - E))
    dl = jnp.pad(dst_s % NPT, (0, EP - E)).astype(jnp.int32)

    bounds = jnp.searchsorted(dst_s, jnp.arange(NT + 1) * NPT).astype(jnp.int32)
    e0 = bounds[:-1]
    e1 = bounds[1:]
    base = (e0 // 8) * 8
    nch = (e1 - base + C - 1) // C
    tinfo = jnp.stack(
        [e0, e1, base, nch] + [jnp.zeros((NT,), jnp.int32)] * 12, axis=1
    ).reshape(-1)

    h = params["node_table"][ent_ids]
    h = jnp.pad(h, ((0, N - h.shape[0]), (0, 0)))

    rtab = _rel_proj(params["rel_table"], params["layers"][0]["Wr"])

    for li, p in enumerate(params["layers"]):
        wstack = jnp.concatenate([p["Wq"], p["Wk"], p["Wv"], p["Wres"]],
                                 axis=1)
        if li == 0:
            h = _layer(h, wstack, src_s, dl, tinfo, EP, rel_s, rtab)
        else:
            h = _layer(h, wstack, src_s, dl, tinfo, EP)

    offsets = jnp.arange(8) * (NODES // 8)
    return h[offsets]


# speedup vs baseline: 40.5172x; 1.2398x over previous
import functools
import jax
import jax.numpy as jnp
import numpy as np
from jax import lax
from jax.experimental import pallas as pl
from jax.experimental.pallas import tpu as pltpu
from jax.experimental.pallas import tpu_sc as plsc

NODES = 10000
N = 10240
NT = 32
NPT = N // NT
C = 128
HEADS = 8
DH = 16
HID = 128
ALPHA = 0.15
HOPS = 4
SQRT_DH = 4.0

_MESH = plsc.VectorSubcoreMesh(core_axis_name="c", subcore_axis_name="s")


def _wid():
    return lax.axis_index("s") * 2 + lax.axis_index("c")


def _m8(x):
    return pl.multiple_of(x, 8)


def _tile_info(ti_v):
    row = ti_v[...]
    return row[0], row[1], row[2], row[3]


def _proj_body(x_ref, w_ref, q_ref, k_ref, v_ref, r_ref):
    y = jnp.dot(x_ref[...], w_ref[...], preferred_element_type=jnp.float32)
    q_ref[...] = y[:, 0:128]
    k_ref[...] = y[:, 128:256]
    v_ref[...] = y[:, 256:384]
    r_ref[...] = y[:, 384:512]


def _proj(h, wstack):
    grid = (N // 256,)
    out = jax.ShapeDtypeStruct((N, HID), jnp.float32)
    return pl.pallas_call(
        _proj_body,
        grid=grid,
        in_specs=[
            pl.BlockSpec((256, HID), lambda i: (i, 0)),
            pl.BlockSpec((HID, 512), lambda i: (0, 0)),
        ],
        out_specs=[pl.BlockSpec((256, HID), lambda i: (i, 0))] * 4,
        out_shape=[out] * 4,
    )(h, wstack)


def _rel_body(t_ref, w_ref, o_ref):
    o_ref[...] = jnp.dot(t_ref[...], w_ref[...],
                         preferred_element_type=jnp.float32)


def _rel_proj(rel_table, wr):
    t = jnp.pad(rel_table, ((0, 0), (0, 112)))
    w = jnp.pad(wr, ((0, 112), (0, 0)))
    return pl.pallas_call(
        _rel_body,
        out_shape=jax.ShapeDtypeStruct((16, HID), jnp.float32),
    )(t, w)


def _combine_body(f_ref, r_ref, o_ref):
    x = f_ref[...] + r_ref[...]
    o_ref[...] = jnp.where(x > 0, x, jnp.exp(jnp.minimum(x, 0.0)) - 1.0)


def _combine(feat, res):
    return pl.pallas_call(
        _combine_body,
        grid=(N // 256,),
        in_specs=[pl.BlockSpec((256, HID), lambda i: (i, 0))] * 2,
        out_specs=pl.BlockSpec((256, HID), lambda i: (i, 0)),
        out_shape=jax.ShapeDtypeStruct((N, HID), jnp.float32),
    )(feat, res)


@functools.lru_cache(maxsize=None)
def _make_scores(EP, with_rel):
    scratch = [
        pltpu.VMEM((16,), jnp.int32),
        pltpu.VMEM((C,), jnp.int32),
        pltpu.VMEM((C + 16,), jnp.int32),
        pltpu.VMEM((C + 16,), jnp.int32),
        pltpu.VMEM((C, HID), jnp.float32),
        pltpu.VMEM((C * 16,), jnp.float32),
        pltpu.VMEM((NPT, HID), jnp.float32),
        pltpu.VMEM((16, HID), jnp.float32),
        pltpu.VMEM((NPT * 16,), jnp.float32),
        pltpu.SemaphoreType.DMA,
    ]

    @functools.partial(
        pl.kernel,
        out_type=[
            jax.ShapeDtypeStruct((EP * 16,), jnp.float32),
            jax.ShapeDtypeStruct((N * 16,), jnp.float32),
        ],
        mesh=_MESH,
        scratch_types=scratch,
    )
    def scores_kernel(ti_hbm, src_hbm, dl_hbm, rel_hbm, q_hbm, k_hbm, r_hbm,
                      sc_hbm, mx_hbm,
                      ti_v, idx_v, dl_v, rel_v, rows_v, sc_v, q_v, r_v, mx_v,
                      sem):
        w = _wid()
        pltpu.sync_copy(ti_hbm.at[pl.ds(_m8(w * 16), 16)], ti_v)
        e0, e1, base, nch = _tile_info(ti_v)
        pltpu.sync_copy(q_hbm.at[pl.ds(_m8(w * NPT), NPT), :], q_v)
        pltpu.sync_copy(r_hbm, r_v)

        neg = jnp.full((16,), -3.0e38, jnp.float32)

        def zrow(i, _):
            mx_v[pl.ds(i * 16, 16)] = neg
            return 0
        lax.fori_loop(0, NPT, zrow, 0)

        def chunk(i, _):
            cs = base + i * C
            pltpu.sync_copy(src_hbm.at[pl.ds(_m8(cs), C)], idx_v)
            pltpu.sync_copy(dl_hbm.at[pl.ds(_m8(cs), C)], dl_v.at[pl.ds(0, C)])
            if with_rel:
                pltpu.sync_copy(rel_hbm.at[pl.ds(_m8(cs), C)],
                                rel_v.at[pl.ds(0, C)])
            pltpu.async_copy(k_hbm.at[idx_v], rows_v, sem).wait()

            lo = jnp.maximum(e0 - cs, 0)
            hi = jnp.minimum(e1 - cs, C)

            def edge(e, _):
                dl = dl_v[pl.ds(e, 16)][0]
                srow = jnp.zeros((16,), jnp.float32)
                iota = lax.iota(jnp.int32, 16)
                if with_rel:
                    rel = rel_v[pl.ds(e, 16)][0]
                for j in range(HEADS):
                    kj = rows_v[e, pl.ds(16 * j, 16)]
                    if with_rel:
                        kj = kj + r_v[rel, pl.ds(16 * j, 16)]
                    qj = q_v[dl, pl.ds(16 * j, 16)]
                    p = qj * kj
                    for sh in (1, 2, 4, 8):
                        p = p + jnp.take_along_axis(
                            p, jnp.bitwise_xor(iota, sh), axis=0)
                    srow = jnp.where(iota == j, p, srow)
                srow = srow * (1.0 / SQRT_DH)
                srow = jnp.where(srow > 0, srow, 0.2 * srow)
                sc_v[pl.ds(e * 16, 16)] = srow
                m = mx_v[pl.ds(dl * 16, 16)]
                mx_v[pl.ds(dl * 16, 16)] = jnp.maximum(m, srow)
                return 0

            lax.fori_loop(lo, hi, edge, 0)

            full = jnp.logical_and(cs >= e0, cs + C <= e1)

            @pl.when(full)
            def _():
                pltpu.sync_copy(sc_v, sc_hbm.at[pl.ds(_m8(cs * 16), C * 16)])

            @pl.when(jnp.logical_not(full))
            def _():
                def wr(e, _):
                    pltpu.sync_copy(sc_v.at[pl.ds(e * 16, 16)],
                                    sc_hbm.at[pl.ds(_m8((cs + e) * 16), 16)])
                    return 0
                lax.fori_loop(lo, hi, wr, 0)
            return 0

        lax.fori_loop(0, nch, chunk, 0)
        pltpu.sync_copy(mx_v, mx_hbm.at[pl.ds(_m8(w * NPT * 16), NPT * 16)])

    return scores_kernel


@functools.lru_cache(maxsize=None)
def _make_expsum(EP):
    scratch = [
        pltpu.VMEM((16,), jnp.int32),
        pltpu.VMEM((C + 16,), jnp.int32),
        pltpu.VMEM((C * 16,), jnp.float32),
        pltpu.VMEM((NPT * 16,), jnp.float32),
        pltpu.VMEM((NPT * 16,), jnp.float32),
    ]

    @functools.partial(
        pl.kernel,
        out_type=[
            jax.ShapeDtypeStruct((EP * 16,), jnp.float32),
            jax.ShapeDtypeStruct((N * 16,), jnp.float32),
        ],
        mesh=_MESH,
        scratch_types=scratch,
    )
    def expsum_kernel(ti_hbm, dl_hbm, sc_hbm, mx_hbm, ev_hbm, inv_hbm,
                      ti_v, dl_v, sc_v, mx_v, st_v):
        w = _wid()
        pltpu.sync_copy(ti_hbm.at[pl.ds(_m8(w * 16), 16)], ti_v)
        e0, e1, base, nch = _tile_info(ti_v)
        pltpu.sync_copy(mx_hbm.at[pl.ds(_m8(w * NPT * 16), NPT * 16)], mx_v)

        zero = jnp.zeros((16,), jnp.float32)

        def zrow(i, _):
            st_v[pl.ds(i * 16, 16)] = zero
            return 0
        lax.fori_loop(0, NPT, zrow, 0)

        def chunk(i, _):
            cs = base + i * C
            pltpu.sync_copy(dl_hbm.at[pl.ds(_m8(cs), C)], dl_v.at[pl.ds(0, C)])
            pltpu.sync_copy(sc_hbm.at[pl.ds(_m8(cs * 16), C * 16)], sc_v)

            lo = jnp.maximum(e0 - cs, 0)
            hi = jnp.minimum(e1 - cs, C)

            def edge(e, _):
                dl = dl_v[pl.ds(e, 16)][0]
                srow = sc_v[pl.ds(e * 16, 16)]
                m = mx_v[pl.ds(dl * 16, 16)]
                ex = jnp.exp(srow - m)
                sc_v[pl.ds(e * 16, 16)] = ex
                st_v[pl.ds(dl * 16, 16)] += ex
                return 0

            lax.fori_loop(lo, hi, edge, 0)

            full = jnp.logical_and(cs >= e0, cs + C <= e1)

            @pl.when(full)
            def _():
                pltpu.sync_copy(sc_v, ev_hbm.at[pl.ds(_m8(cs * 16), C * 16)])

            @pl.when(jnp.logical_not(full))
            def _():
                def wr(e, _):
                    pltpu.sync_copy(sc_v.at[pl.ds(e * 16, 16)],
                                    ev_hbm.at[pl.ds(_m8((cs + e) * 16), 16)])
                    return 0
                lax.fori_loop(lo, hi, wr, 0)
            return 0

        lax.fori_loop(0, nch, chunk, 0)

        def inv(i, _):
            s = st_v[pl.ds(i * 16, 16)]
            st_v[pl.ds(i * 16, 16)] = 1.0 / (s + 1e-9)
            return 0
        lax.fori_loop(0, NPT, inv, 0)
        pltpu.sync_copy(st_v, inv_hbm.at[pl.ds(_m8(w * NPT * 16), NPT * 16)])

    return expsum_kernel


@functools.lru_cache(maxsize=None)
def _make_hop(EP):
    scratch = [
        pltpu.VMEM((16,), jnp.int32),
        pltpu.VMEM((C,), jnp.int32),
        pltpu.VMEM((C,), jnp.int32),
        pltpu.VMEM((C + 16,), jnp.int32),
        pltpu.VMEM((C + 16,), jnp.int32),
        pltpu.VMEM((C * 16,), jnp.float32),
        pltpu.VMEM((C * 16,), jnp.float32),
        pltpu.VMEM((C, HID), jnp.float32),
        pltpu.VMEM((C, HID), jnp.float32),
        pltpu.VMEM((NPT, HID), jnp.float32),
        pltpu.VMEM((NPT, HID), jnp.float32),
        pltpu.VMEM((NPT * 16,), jnp.float32),
        pltpu.SemaphoreType.DMA,
        pltpu.SemaphoreType.DMA,
        pltpu.SemaphoreType.DMA,
        pltpu.SemaphoreType.DMA,
        pltpu.SemaphoreType.DMA,
        pltpu.SemaphoreType.DMA,
        pltpu.SemaphoreType.DMA,
        pltpu.SemaphoreType.DMA,
    ]

    @functools.partial(
        pl.kernel,
        out_type=jax.ShapeDtypeStruct((N, HID), jnp.float32),
        mesh=_MESH,
        scratch_types=scratch,
    )
    def hop_kernel(ti_hbm, src_hbm, dl_hbm, ev_hbm, inv_hbm, feat_hbm, v_hbm,
                   out_hbm,
                   ti_v, idx0, idx1, dl0, dl1, ev0, ev1, rows0, rows1,
                   acc_v, v0_v, inv_v, gs0, gs1, is0, is1, ds0, ds1, es0, es1):
        w = _wid()
        pltpu.sync_copy(ti_hbm.at[pl.ds(_m8(w * 16), 16)], ti_v)
        e0, e1, base, nch = _tile_info(ti_v)
        pltpu.sync_copy(inv_hbm.at[pl.ds(_m8(w * NPT * 16), NPT * 16)], inv_v)
        pltpu.sync_copy(v_hbm.at[pl.ds(_m8(w * NPT), NPT), :], v0_v)

        idxb, dlb, evb, rowsb = (idx0, idx1), (dl0, dl1), (ev0, ev1), (rows0, rows1)
        isem, gsem, dsem, esem = (is0, is1), (gs0, gs1), (ds0, ds1), (es0, es1)

        def cs_of(ci):
            return _m8(base + ci * C)

        def idx_copy(ci, b):
            return pltpu.make_async_copy(
                src_hbm.at[pl.ds(cs_of(ci), C)], idxb[b], isem[b])

        def gather_copy(b):
            return pltpu.make_async_copy(feat_hbm.at[idxb[b]], rowsb[b], gsem[b])

        def dl_copy(ci, b):
            return pltpu.make_async_copy(
                dl_hbm.at[pl.ds(cs_of(ci), C)], dlb[b].at[pl.ds(0, C)], dsem[b])

        def ev_copy(ci, b):
            return pltpu.make_async_copy(
                ev_hbm.at[pl.ds(_m8(cs_of(ci) * 16), C * 16)], evb[b], esem[b])

        zero = jnp.zeros((16,), jnp.float32)

        def zrow(i, _):
            for j in range(HEADS):
                acc_v[i, pl.ds(16 * j, 16)] = zero
            return 0
        lax.fori_loop(0, NPT, zrow, 0)

        @pl.when(nch > 0)
        def _():
            idx_copy(0, 0).start()
            idx_copy(0, 0).wait()
            gather_copy(0).start()
            dl_copy(0, 0).start()
            ev_copy(0, 0).start()

        @pl.when(nch > 1)
        def _():
            idx_copy(1, 1).start()

        def process(ci, b):
            cs = cs_of(ci)
            lo = jnp.maximum(e0 - cs, 0)
            hi = jnp.minimum(e1 - cs, C)
            dl_v, ev_v, rows_v = dlb[b], evb[b], rowsb[b]

            def edge(e, _):
                dl = dl_v[pl.ds(e, 16)][0]
                evrow = ev_v[pl.ds(e * 16, 16)]
                for j in range(HEADS):
                    evj = jnp.full((16,), evrow[j], jnp.float32)
                    r = rows_v[e, pl.ds(16 * j, 16)]
                    acc_v[dl, pl.ds(16 * j, 16)] += r * evj
                return 0

            lax.fori_loop(lo, hi, edge, 0)

        def pair(p, _):
            for b in (0, 1):
                ci = 2 * p + b
                nb = 1 - b

                @pl.when(ci < nch)
                def _():
                    @pl.when(ci + 1 < nch)
                    def _():
                        idx_copy(ci + 1, nb).wait()
                        gather_copy(nb).start()
                        dl_copy(ci + 1, nb).start()
                        ev_copy(ci + 1, nb).start()

                    gather_copy(b).wait()
                    dl_copy(ci, b).wait()
                    ev_copy(ci, b).wait()

                    @pl.when(ci + 2 < nch)
                    def _():
                        idx_copy(ci + 2, b).start()

                    process(ci, b)
            return 0

        lax.fori_loop(0, (nch + 1) // 2, pair, 0)

        def finish(i, _):
            invrow = inv_v[pl.ds(i * 16, 16)]
            for j in range(HEADS):
                ivj = jnp.full((16,), invrow[j], jnp.float32)
                a = acc_v[i, pl.ds(16 * j, 16)]
                f0 = v0_v[i, pl.ds(16 * j, 16)]
                acc_v[i, pl.ds(16 * j, 16)] = (1.0 - ALPHA) * a * ivj + ALPHA * f0
            return 0
        lax.fori_loop(0, NPT, finish, 0)
        pltpu.sync_copy(acc_v, out_hbm.at[pl.ds(_m8(w * NPT), NPT), :])

    return hop_kernel


def _layer(h, wstack, src_s, dl, tinfo, EP, rel_s=None, rtab=None):
    q, k, v, res = _proj(h, wstack)
    with_rel = rel_s is not None
    if not with_rel:
        rtab = jnp.zeros((16, HID), jnp.float32)
        rel_s = jnp.zeros((EP,), jnp.int32)
    scores_k = _make_scores(EP, with_rel)
    sc, mx = scores_k(tinfo, src_s, dl, rel_s, q, k, rtab)
    expsum_k = _make_expsum(EP)
    ev, inv = expsum_k(tinfo, dl, sc, mx)
    hop_k = _make_hop(EP)
    feat = v
    for _ in range(HOPS):
        feat = hop_k(tinfo, src_s, dl, ev, inv, feat, v)
    return _combine(feat, res)


def kernel(ent_ids, rel_ids, edge_index, params):
    src = edge_index[0].astype(jnp.int32)
    dst = edge_index[1].astype(jnp.int32)
    E = src.shape[0]
    EP = E + C

    perm = jnp.argsort(dst)
    src_s = jnp.pad(src[perm], (0, EP - E))
    dst_s = dst[perm]
    rel_s = jnp.pad(rel_ids[perm].astype(jnp.int32), (0, EP - E))
    dl = jnp.pad(dst_s % NPT, (0, EP - E)).astype(jnp.int32)

    bounds = jnp.searchsorted(dst_s, jnp.arange(NT + 1) * NPT).astype(jnp.int32)
    e0 = bounds[:-1]
    e1 = bounds[1:]
    base = (e0 // 8) * 8
    nch = (e1 - base + C - 1) // C
    tinfo = jnp.stack(
        [e0, e1, base, nch] + [jnp.zeros((NT,), jnp.int32)] * 12, axis=1
    ).reshape(-1)

    h = params["node_table"][ent_ids]
    h = jnp.pad(h, ((0, N - h.shape[0]), (0, 0)))

    rtab = _rel_proj(params["rel_table"], params["layers"][0]["Wr"])

    for li, p in enumerate(params["layers"]):
        wstack = jnp.concatenate([p["Wq"], p["Wk"], p["Wv"], p["Wres"]],
                                 axis=1)
        if li == 0:
            h = _layer(h, wstack, src_s, dl, tinfo, EP, rel_s, rtab)
        else:
            h = _layer(h, wstack, src_s, dl, tinfo, EP)

    offsets = jnp.arange(8) * (NODES // 8)
    return h[offsets]


# speedup vs baseline: 43.5947x; 1.0760x over previous
import functools
import jax
import jax.numpy as jnp
import numpy as np
from jax import lax
from jax.experimental import pallas as pl
from jax.experimental.pallas import tpu as pltpu
from jax.experimental.pallas import tpu_sc as plsc

NODES = 10000
N = 10240
NT = 32
NPT = N // NT
C = 128
HEADS = 8
DH = 16
HID = 128
ALPHA = 0.15
HOPS = 4
SQRT_DH = 4.0

_MESH = plsc.VectorSubcoreMesh(core_axis_name="c", subcore_axis_name="s")


def _wid():
    return lax.axis_index("s") * 2 + lax.axis_index("c")


def _m8(x):
    return pl.multiple_of(x, 8)


def _tile_info(ti_v):
    row = ti_v[...]
    return row[0], row[1], row[2], row[3]


def _proj_body(x_ref, w_ref, q_ref, k_ref, v_ref, r_ref):
    y = jnp.dot(x_ref[...], w_ref[...], preferred_element_type=jnp.float32)
    q_ref[...] = y[:, 0:128]
    k_ref[...] = y[:, 128:256]
    v_ref[...] = y[:, 256:384]
    r_ref[...] = y[:, 384:512]


def _proj(h, wstack):
    grid = (N // 256,)
    out = jax.ShapeDtypeStruct((N, HID), jnp.float32)
    return pl.pallas_call(
        _proj_body,
        grid=grid,
        in_specs=[
            pl.BlockSpec((256, HID), lambda i: (i, 0)),
            pl.BlockSpec((HID, 512), lambda i: (0, 0)),
        ],
        out_specs=[pl.BlockSpec((256, HID), lambda i: (i, 0))] * 4,
        out_shape=[out] * 4,
    )(h, wstack)


def _rel_body(t_ref, w_ref, o_ref):
    o_ref[...] = jnp.dot(t_ref[...], w_ref[...],
                         preferred_element_type=jnp.float32)


def _rel_proj(rel_table, wr):
    t = jnp.pad(rel_table, ((0, 0), (0, 112)))
    w = jnp.pad(wr, ((0, 112), (0, 0)))
    return pl.pallas_call(
        _rel_body,
        out_shape=jax.ShapeDtypeStruct((16, HID), jnp.float32),
    )(t, w)


def _combine_body(f_ref, r_ref, o_ref):
    x = f_ref[...] + r_ref[...]
    o_ref[...] = jnp.where(x > 0, x, jnp.exp(jnp.minimum(x, 0.0)) - 1.0)


def _combine(feat, res):
    return pl.pallas_call(
        _combine_body,
        grid=(N // 256,),
        in_specs=[pl.BlockSpec((256, HID), lambda i: (i, 0))] * 2,
        out_specs=pl.BlockSpec((256, HID), lambda i: (i, 0)),
        out_shape=jax.ShapeDtypeStruct((N, HID), jnp.float32),
    )(feat, res)


@functools.lru_cache(maxsize=None)
def _make_scores(EP, with_rel):
    scratch = [
        pltpu.VMEM((16,), jnp.int32),
        pltpu.VMEM((C,), jnp.int32),
        pltpu.VMEM((C,), jnp.int32),
        pltpu.VMEM((C + 16,), jnp.int32),
        pltpu.VMEM((C + 16,), jnp.int32),
        pltpu.VMEM((C + 16,), jnp.int32),
        pltpu.VMEM((C + 16,), jnp.int32),
        pltpu.VMEM((C, HID), jnp.float32),
        pltpu.VMEM((C, HID), jnp.float32),
        pltpu.VMEM((C * 16,), jnp.float32),
        pltpu.VMEM((NPT, HID), jnp.float32),
        pltpu.VMEM((16, HID), jnp.float32),
        pltpu.VMEM((NPT * 16,), jnp.float32),
        pltpu.SemaphoreType.DMA,
        pltpu.SemaphoreType.DMA,
        pltpu.SemaphoreType.DMA,
        pltpu.SemaphoreType.DMA,
        pltpu.SemaphoreType.DMA,
        pltpu.SemaphoreType.DMA,
        pltpu.SemaphoreType.DMA,
        pltpu.SemaphoreType.DMA,
    ]

    @functools.partial(
        pl.kernel,
        out_type=[
            jax.ShapeDtypeStruct((EP * 16,), jnp.float32),
            jax.ShapeDtypeStruct((N * 16,), jnp.float32),
        ],
        mesh=_MESH,
        scratch_types=scratch,
    )
    def scores_kernel(ti_hbm, src_hbm, dl_hbm, rel_hbm, q_hbm, k_hbm, r_hbm,
                      sc_hbm, mx_hbm,
                      ti_v, idx0, idx1, dl0, dl1, rl0, rl1, rows0, rows1,
                      sc_v, q_v, r_v, mx_v,
                      gs0, gs1, is0, is1, ds0, ds1, rs0, rs1):
        w = _wid()
        pltpu.sync_copy(ti_hbm.at[pl.ds(_m8(w * 16), 16)], ti_v)
        e0, e1, base, nch = _tile_info(ti_v)
        pltpu.sync_copy(q_hbm.at[pl.ds(_m8(w * NPT), NPT), :], q_v)
        pltpu.sync_copy(r_hbm, r_v)

        idxb, dlb, rlb, rowsb = (idx0, idx1), (dl0, dl1), (rl0, rl1), (rows0, rows1)
        isem, gsem, dsem, rsem = (is0, is1), (gs0, gs1), (ds0, ds1), (rs0, rs1)

        def cs_of(ci):
            return _m8(base + ci * C)

        def idx_copy(ci, b):
            return pltpu.make_async_copy(
                src_hbm.at[pl.ds(cs_of(ci), C)], idxb[b], isem[b])

        def gather_copy(b):
            return pltpu.make_async_copy(k_hbm.at[idxb[b]], rowsb[b], gsem[b])

        def meta_start(ci, b):
            pltpu.make_async_copy(
                dl_hbm.at[pl.ds(cs_of(ci), C)], dlb[b].at[pl.ds(0, C)],
                dsem[b]).start()
            if with_rel:
                pltpu.make_async_copy(
                    rel_hbm.at[pl.ds(cs_of(ci), C)], rlb[b].at[pl.ds(0, C)],
                    rsem[b]).start()

        def meta_wait(ci, b):
            pltpu.make_async_copy(
                dl_hbm.at[pl.ds(cs_of(ci), C)], dlb[b].at[pl.ds(0, C)],
                dsem[b]).wait()
            if with_rel:
                pltpu.make_async_copy(
                    rel_hbm.at[pl.ds(cs_of(ci), C)], rlb[b].at[pl.ds(0, C)],
                    rsem[b]).wait()

        neg = jnp.full((16,), -3.0e38, jnp.float32)

        def zrow(i, _):
            mx_v[pl.ds(i * 16, 16)] = neg
            return 0
        lax.fori_loop(0, NPT, zrow, 0)

        @pl.when(nch > 0)
        def _():
            idx_copy(0, 0).start()
            idx_copy(0, 0).wait()
            gather_copy(0).start()
            meta_start(0, 0)

        @pl.when(nch > 1)
        def _():
            idx_copy(1, 1).start()

        def process(ci, b):
            cs = cs_of(ci)
            lo = jnp.maximum(e0 - cs, 0)
            hi = jnp.minimum(e1 - cs, C)
            dl_v, rel_v, rows_v = dlb[b], rlb[b], rowsb[b]

            def edge(e, _):
                dl = dl_v[pl.ds(e, 16)][0]
                srow = jnp.zeros((16,), jnp.float32)
                iota = lax.iota(jnp.int32, 16)
                if with_rel:
                    rel = rel_v[pl.ds(e, 16)][0]
                for j in range(HEADS):
                    kj = rows_v[e, pl.ds(16 * j, 16)]
                    if with_rel:
                        kj = kj + r_v[rel, pl.ds(16 * j, 16)]
                    qj = q_v[dl, pl.ds(16 * j, 16)]
                    p = qj * kj
                    for sh in (1, 2, 4, 8):
                        p = p + jnp.take_along_axis(
                            p, jnp.bitwise_xor(iota, sh), axis=0)
                    srow = jnp.where(iota == j, p, srow)
                srow = srow * (1.0 / SQRT_DH)
                srow = jnp.where(srow > 0, srow, 0.2 * srow)
                sc_v[pl.ds(e * 16, 16)] = srow
                m = mx_v[pl.ds(dl * 16, 16)]
                mx_v[pl.ds(dl * 16, 16)] = jnp.maximum(m, srow)
                return 0

            lax.fori_loop(lo, hi, edge, 0)

            full = jnp.logical_and(cs >= e0, cs + C <= e1)

            @pl.when(full)
            def _():
                pltpu.sync_copy(sc_v, sc_hbm.at[pl.ds(_m8(cs * 16), C * 16)])

            @pl.when(jnp.logical_not(full))
            def _():
                def wr(e, _):
                    pltpu.sync_copy(sc_v.at[pl.ds(e * 16, 16)],
                                    sc_hbm.at[pl.ds(_m8((cs + e) * 16), 16)])
                    return 0
                lax.fori_loop(lo, hi, wr, 0)

        def pair(p, _):
            for b in (0, 1):
                ci = 2 * p + b
                nb = 1 - b

                @pl.when(ci < nch)
                def _():
                    @pl.when(ci + 1 < nch)
                    def _():
                        idx_copy(ci + 1, nb).wait()
                        gather_copy(nb).start()
                        meta_start(ci + 1, nb)

                    gather_copy(b).wait()
                    meta_wait(ci, b)

                    @pl.when(ci + 2 < nch)
                    def _():
                        idx_copy(ci + 2, b).start()

                    process(ci, b)
            return 0

        lax.fori_loop(0, (nch + 1) // 2, pair, 0)
        pltpu.sync_copy(mx_v, mx_hbm.at[pl.ds(_m8(w * NPT * 16), NPT * 16)])

    return scores_kernel


@functools.lru_cache(maxsize=None)
def _make_expsum(EP):
    scratch = [
        pltpu.VMEM((16,), jnp.int32),
        pltpu.VMEM((C + 16,), jnp.int32),
        pltpu.VMEM((C + 16,), jnp.int32),
        pltpu.VMEM((C * 16,), jnp.float32),
        pltpu.VMEM((C * 16,), jnp.float32),
        pltpu.VMEM((NPT * 16,), jnp.float32),
        pltpu.VMEM((NPT * 16,), jnp.float32),
        pltpu.SemaphoreType.DMA,
        pltpu.SemaphoreType.DMA,
        pltpu.SemaphoreType.DMA,
        pltpu.SemaphoreType.DMA,
    ]

    @functools.partial(
        pl.kernel,
        out_type=[
            jax.ShapeDtypeStruct((EP * 16,), jnp.float32),
            jax.ShapeDtypeStruct((N * 16,), jnp.float32),
        ],
        mesh=_MESH,
        scratch_types=scratch,
    )
    def expsum_kernel(ti_hbm, dl_hbm, sc_hbm, mx_hbm, ev_hbm, inv_hbm,
                      ti_v, dl0, dl1, sc0, sc1, mx_v, st_v,
                      ds0, ds1, ss0, ss1):
        w = _wid()
        pltpu.sync_copy(ti_hbm.at[pl.ds(_m8(w * 16), 16)], ti_v)
        e0, e1, base, nch = _tile_info(ti_v)
        pltpu.sync_copy(mx_hbm.at[pl.ds(_m8(w * NPT * 16), NPT * 16)], mx_v)

        dlb, scb = (dl0, dl1), (sc0, sc1)
        dsem, ssem = (ds0, ds1), (ss0, ss1)

        def cs_of(ci):
            return _m8(base + ci * C)

        def dl_copy(ci, b):
            return pltpu.make_async_copy(
                dl_hbm.at[pl.ds(cs_of(ci), C)], dlb[b].at[pl.ds(0, C)],
                dsem[b])

        def sc_copy(ci, b):
            return pltpu.make_async_copy(
                sc_hbm.at[pl.ds(_m8(cs_of(ci) * 16), C * 16)], scb[b],
                ssem[b])

        zero = jnp.zeros((16,), jnp.float32)

        def zrow(i, _):
            st_v[pl.ds(i * 16, 16)] = zero
            return 0
        lax.fori_loop(0, NPT, zrow, 0)

        @pl.when(nch > 0)
        def _():
            dl_copy(0, 0).start()
            sc_copy(0, 0).start()

        def process(ci, b):
            cs = cs_of(ci)
            lo = jnp.maximum(e0 - cs, 0)
            hi = jnp.minimum(e1 - cs, C)
            dl_v, sc_v = dlb[b], scb[b]

            def edge(e, _):
                dl = dl_v[pl.ds(e, 16)][0]
                srow = sc_v[pl.ds(e * 16, 16)]
                m = mx_v[pl.ds(dl * 16, 16)]
                ex = jnp.exp(srow - m)
                sc_v[pl.ds(e * 16, 16)] = ex
                st_v[pl.ds(dl * 16, 16)] += ex
                return 0

            lax.fori_loop(lo, hi, edge, 0)

            full = jnp.logical_and(cs >= e0, cs + C <= e1)

            @pl.when(full)
            def _():
                pltpu.sync_copy(sc_v, ev_hbm.at[pl.ds(_m8(cs * 16), C * 16)])

            @pl.when(jnp.logical_not(full))
            def _():
                def wr(e, _):
                    pltpu.sync_copy(sc_v.at[pl.ds(e * 16, 16)],
                                    ev_hbm.at[pl.ds(_m8((cs + e) * 16), 16)])
                    return 0
                lax.fori_loop(lo, hi, wr, 0)

        def pair(p, _):
            for b in (0, 1):
                ci = 2 * p + b
                nb = 1 - b

                @pl.when(ci < nch)
                def _():
                    @pl.when(ci + 1 < nch)
                    def _():
                        dl_copy(ci + 1, nb).start()
                        sc_copy(ci + 1, nb).start()

                    dl_copy(ci, b).wait()
                    sc_copy(ci, b).wait()
                    process(ci, b)
            return 0

        lax.fori_loop(0, (nch + 1) // 2, pair, 0)

        def inv(i, _):
            s = st_v[pl.ds(i * 16, 16)]
            st_v[pl.ds(i * 16, 16)] = 1.0 / (s + 1e-9)
            return 0
        lax.fori_loop(0, NPT, inv, 0)
        pltpu.sync_copy(st_v, inv_hbm.at[pl.ds(_m8(w * NPT * 16), NPT * 16)])

    return expsum_kernel


@functools.lru_cache(maxsize=None)
def _make_hop(EP):
    scratch = [
        pltpu.VMEM((16,), jnp.int32),
        pltpu.VMEM((C,), jnp.int32),
        pltpu.VMEM((C,), jnp.int32),
        pltpu.VMEM((C + 16,), jnp.int32),
        pltpu.VMEM((C + 16,), jnp.int32),
        pltpu.VMEM((C * 16,), jnp.float32),
        pltpu.VMEM((C * 16,), jnp.float32),
        pltpu.VMEM((C, HID), jnp.float32),
        pltpu.VMEM((C, HID), jnp.float32),
        pltpu.VMEM((NPT, HID), jnp.float32),
        pltpu.VMEM((NPT, HID), jnp.float32),
        pltpu.VMEM((NPT * 16,), jnp.float32),
        pltpu.SemaphoreType.DMA,
        pltpu.SemaphoreType.DMA,
        pltpu.SemaphoreType.DMA,
        pltpu.SemaphoreType.DMA,
        pltpu.SemaphoreType.DMA,
        pltpu.SemaphoreType.DMA,
        pltpu.SemaphoreType.DMA,
        pltpu.SemaphoreType.DMA,
    ]

    @functools.partial(
        pl.kernel,
        out_type=jax.ShapeDtypeStruct((N, HID), jnp.float32),
        mesh=_MESH,
        scratch_types=scratch,
    )
    def hop_kernel(ti_hbm, src_hbm, dl_hbm, ev_hbm, inv_hbm, feat_hbm, v_hbm,
                   out_hbm,
                   ti_v, idx0, idx1, dl0, dl1, ev0, ev1, rows0, rows1,
                   acc_v, v0_v, inv_v, gs0, gs1, is0, is1, ds0, ds1, es0, es1):
        w = _wid()
        pltpu.sync_copy(ti_hbm.at[pl.ds(_m8(w * 16), 16)], ti_v)
        e0, e1, base, nch = _tile_info(ti_v)
        pltpu.sync_copy(inv_hbm.at[pl.ds(_m8(w * NPT * 16), NPT * 16)], inv_v)
        pltpu.sync_copy(v_hbm.at[pl.ds(_m8(w * NPT), NPT), :], v0_v)

        idxb, dlb, evb, rowsb = (idx0, idx1), (dl0, dl1), (ev0, ev1), (rows0, rows1)
        isem, gsem, dsem, esem = (is0, is1), (gs0, gs1), (ds0, ds1), (es0, es1)

        def cs_of(ci):
            return _m8(base + ci * C)

        def idx_copy(ci, b):
            return pltpu.make_async_copy(
                src_hbm.at[pl.ds(cs_of(ci), C)], idxb[b], isem[b])

        def gather_copy(b):
            return pltpu.make_async_copy(feat_hbm.at[idxb[b]], rowsb[b], gsem[b])

        def dl_copy(ci, b):
            return pltpu.make_async_copy(
                dl_hbm.at[pl.ds(cs_of(ci), C)], dlb[b].at[pl.ds(0, C)], dsem[b])

        def ev_copy(ci, b):
            return pltpu.make_async_copy(
                ev_hbm.at[pl.ds(_m8(cs_of(ci) * 16), C * 16)], evb[b], esem[b])

        zero = jnp.zeros((16,), jnp.float32)

        def zrow(i, _):
            for j in range(HEADS):
                acc_v[i, pl.ds(16 * j, 16)] = zero
            return 0
        lax.fori_loop(0, NPT, zrow, 0)

        @pl.when(nch > 0)
        def _():
            idx_copy(0, 0).start()
            idx_copy(0, 0).wait()
            gather_copy(0).start()
            dl_copy(0, 0).start()
            ev_copy(0, 0).start()

        @pl.when(nch > 1)
        def _():
            idx_copy(1, 1).start()

        def process(ci, b):
            cs = cs_of(ci)
            lo = jnp.maximum(e0 - cs, 0)
            hi = jnp.minimum(e1 - cs, C)
            dl_v, ev_v, rows_v = dlb[b], evb[b], rowsb[b]

            def edge(e, _):
                dl = dl_v[pl.ds(e, 16)][0]
                evrow = ev_v[pl.ds(e * 16, 16)]
                for j in range(HEADS):
                    evj = jnp.full((16,), evrow[j], jnp.float32)
                    r = rows_v[e, pl.ds(16 * j, 16)]
                    acc_v[dl, pl.ds(16 * j, 16)] += r * evj
                return 0

            lax.fori_loop(lo, hi, edge, 0)

        def pair(p, _):
            for b in (0, 1):
                ci = 2 * p + b
                nb = 1 - b

                @pl.when(ci < nch)
                def _():
                    @pl.when(ci + 1 < nch)
                    def _():
                        idx_copy(ci + 1, nb).wait()
                        gather_copy(nb).start()
                        dl_copy(ci + 1, nb).start()
                        ev_copy(ci + 1, nb).start()

                    gather_copy(b).wait()
                    dl_copy(ci, b).wait()
                    ev_copy(ci, b).wait()

                    @pl.when(ci + 2 < nch)
                    def _():
                        idx_copy(ci + 2, b).start()

                    process(ci, b)
            return 0

        lax.fori_loop(0, (nch + 1) // 2, pair, 0)

        def finish(i, _):
            invrow = inv_v[pl.ds(i * 16, 16)]
            for j in range(HEADS):
                ivj = jnp.full((16,), invrow[j], jnp.float32)
                a = acc_v[i, pl.ds(16 * j, 16)]
                f0 = v0_v[i, pl.ds(16 * j, 16)]
                acc_v[i, pl.ds(16 * j, 16)] = (1.0 - ALPHA) * a * ivj + ALPHA * f0
            return 0
        lax.fori_loop(0, NPT, finish, 0)
        pltpu.sync_copy(acc_v, out_hbm.at[pl.ds(_m8(w * NPT), NPT), :])

    return hop_kernel


def _layer(h, wstack, src_s, dl, tinfo, EP, rel_s=None, rtab=None):
    q, k, v, res = _proj(h, wstack)
    with_rel = rel_s is not None
    if not with_rel:
        rtab = jnp.zeros((16, HID), jnp.float32)
        rel_s = jnp.zeros((EP,), jnp.int32)
    scores_k = _make_scores(EP, with_rel)
    sc, mx = scores_k(tinfo, src_s, dl, rel_s, q, k, rtab)
    expsum_k = _make_expsum(EP)
    ev, inv = expsum_k(tinfo, dl, sc, mx)
    hop_k = _make_hop(EP)
    feat = v
    for _ in range(HOPS):
        feat = hop_k(tinfo, src_s, dl, ev, inv, feat, v)
    return _combine(feat, res)


def kernel(ent_ids, rel_ids, edge_index, params):
    src = edge_index[0].astype(jnp.int32)
    dst = edge_index[1].astype(jnp.int32)
    E = src.shape[0]
    EP = E + C

    perm = jnp.argsort(dst)
    src_s = jnp.pad(src[perm], (0, EP - E))
    dst_s = dst[perm]
    rel_s = jnp.pad(rel_ids[perm].astype(jnp.int32), (0, EP - E))
    dl = jnp.pad(dst_s % NPT, (0, EP - E)).astype(jnp.int32)

    bounds = jnp.searchsorted(dst_s, jnp.arange(NT + 1) * NPT).astype(jnp.int32)
    e0 = bounds[:-1]
    e1 = bounds[1:]
    base = (e0 // 8) * 8
    nch = (e1 - base + C - 1) // C
    tinfo = jnp.stack(
        [e0, e1, base, nch] + [jnp.zeros((NT,), jnp.int32)] * 12, axis=1
    ).reshape(-1)

    h = params["node_table"][ent_ids]
    h = jnp.pad(h, ((0, N - h.shape[0]), (0, 0)))

    rtab = _rel_proj(params["rel_table"], params["layers"][0]["Wr"])

    for li, p in enumerate(params["layers"]):
        wstack = jnp.concatenate([p["Wq"], p["Wk"], p["Wv"], p["Wres"]],
                                 axis=1)
        if li == 0:
            h = _layer(h, wstack, src_s, dl, tinfo, EP, rel_s, rtab)
        else:
            h = _layer(h, wstack, src_s, dl, tinfo, EP)

    offsets = jnp.arange(8) * (NODES // 8)
    return h[offsets]
